# baseline (speedup 1.0000x reference)
import functools

import jax
import jax.numpy as jnp
import numpy as np
from jax import lax
from jax.experimental import pallas as pl
from jax.experimental.pallas import tpu as pltpu
from jax.experimental.pallas import tpu_sc as plsc

_SPACE_DIMS = 3
_B = 16
_ROW_LEN = 2048
_N = _B * _ROW_LEN
_HALF = _ROW_LEN // 2
_CHUNKS = _HALF // 16
_SLAB = _HALF * _SPACE_DIMS

_THRESH_DEG = 1.0


def _deg2dist(deg):
    return 2.0 * np.sin(np.radians(deg) / 2.0)


_THRESH_S2_MIN = float(_deg2dist(10.0 / 3600.0) ** 2)
_THRESH_S2_MAX = float(_deg2dist(_THRESH_DEG) ** 2)
_LOG_THRESH_S2_RANGE = float(np.log(_THRESH_S2_MAX / _THRESH_S2_MIN))
_THRESH_HIT_PROB_POST = 0.95

_LN2 = np.float32(0.6931471805599453)
_F = jnp.float32


def _approx_sqrt(v):
    vc = jnp.maximum(v, _F(1e-20))
    i = lax.bitcast_convert_type(vc, jnp.int32)
    i = jnp.int32(0x5F3759DF) - lax.shift_right_arithmetic(i, 1)
    y = lax.bitcast_convert_type(i, jnp.float32)
    for _ in range(3):
        y = y * (_F(1.5) - _F(0.5) * vc * y * y)
    return vc * y


def _approx_log(x):
    i = lax.bitcast_convert_type(x, jnp.int32)
    e = lax.shift_right_arithmetic(i, 23) - jnp.int32(127)
    mi = jnp.bitwise_or(jnp.bitwise_and(i, jnp.int32(0x007FFFFF)),
                        jnp.int32(0x3F800000))
    m = lax.bitcast_convert_type(mi, jnp.float32)
    big = m > _F(1.4142135)
    m = jnp.where(big, m * _F(0.5), m)
    e = jnp.where(big, e + 1, e)
    s = (m - _F(1.0)) / (m + _F(1.0))
    s2 = s * s
    poly = _F(1.0) + s2 * (_F(1.0 / 3.0) + s2 * (_F(0.2) + s2 * _F(1.0 / 7.0)))
    return e.astype(jnp.float32) * _LN2 + _F(2.0) * s * poly


def _splat(vec16, idx):
    return vec16[jnp.full((16,), idx, dtype=jnp.int32)]


def _lanesum(x):
    lane = jax.lax.iota(jnp.int32, 16)
    for sh in (8, 4, 2, 1):
        x = x + x[jnp.bitwise_and(lane + sh, 15)]
    return x


def _triple_sum(q0, q1, q2):
    lane = jax.lax.iota(jnp.int32, 16)
    out = None
    regs = (q0, q1, q2)
    for t in range(3):
        idx = lane * 3 + t
        lanes = jnp.bitwise_and(idx, 15)
        rv = lax.shift_right_logical(idx, 4)
        g = [r[lanes] for r in regs]
        v = jnp.where(rv == 0, g[0], jnp.where(rv == 1, g[1], g[2]))
        out = v if out is None else out + v
    return out


def _sc_body(up_hbm, uo_hbm, h_hbm, lam_hbm, ts_hbm, parts_hbm,
             vp, vo, vh, vlam, vts, vstage, sem_s, sem_p):
    c = lax.axis_index("c")
    s = lax.axis_index("s")
    w = c * 16 + s
    seg = c * 8 + lax.div(s, 2)

    cp1 = pltpu.async_copy(up_hbm.at[w], vp, sem_s)
    cp2 = pltpu.async_copy(uo_hbm.at[w], vo, sem_s)
    cp3 = pltpu.async_copy(h_hbm, vh, sem_p)
    cp4 = pltpu.async_copy(lam_hbm, vlam, sem_p)
    cp5 = pltpu.async_copy(ts_hbm, vts, sem_p)
    cp5.wait()
    cp4.wait()
    cp3.wait()

    h16 = vh[...]
    lam16 = vlam[...]
    ts16 = vts[...]
    t16 = _F(_THRESH_S2_MIN) * jnp.exp(ts16 * _F(_LOG_THRESH_S2_RANGE))
    a16 = h16 * lam16 / (_F(1.0) - jnp.exp(-lam16))
    thr = _splat(t16, seg)
    inv_thr = _F(1.0) / thr
    lam_b = _splat(lam16, seg)
    a_b = _splat(a16, seg)
    c_b = _F(1.0) - _splat(h16, seg)
    cp2.wait()
    cp1.wait()

    zeros = jnp.zeros((16,), jnp.float32)

    def acc_body(i, carry):
        cnt, ll, lw, den, ht = carry
        off = i * 48
        d0 = vp[pl.ds(off, 16)] - vo[pl.ds(off, 16)]
        d1 = vp[pl.ds(off + 16, 16)] - vo[pl.ds(off + 16, 16)]
        d2 = vp[pl.ds(off + 32, 16)] - vo[pl.ds(off + 32, 16)]
        s2 = _triple_sum(d0 * d0, d1 * d1, d2 * d2)
        close = s2 < thr
        v = s2 * inv_thr
        obs_w = jnp.where(close, jnp.exp(_F(-2.0) * _approx_sqrt(v)), _F(0.0))
        p_hit = a_b * jnp.exp(-lam_b * v)
        p = p_hit + c_b
        log_p = jnp.where(close, _approx_log(p), _F(0.0))
        php = p_hit / p
        phf = jnp.where(jnp.logical_and(close, php > _F(_THRESH_HIT_PROB_POST)),
                        php, _F(0.0))
        return (cnt + jnp.where(close, _F(1.0), _F(0.0)),
                ll + log_p,
                lw + log_p * obs_w,
                den + obs_w,
                ht + phf)

    cnt, ll, lw, den, ht = lax.fori_loop(
        0, _CHUNKS, acc_body, (zeros, zeros, zeros, zeros, zeros))

    lane = jax.lax.iota(jnp.int32, 16)
    stage = jnp.zeros((16,), jnp.float32)
    stage = jnp.where(lane == 0, _lanesum(cnt), stage)
    stage = jnp.where(lane == 1, _lanesum(ll), stage)
    stage = jnp.where(lane == 2, _lanesum(lw), stage)
    stage = jnp.where(lane == 3, _lanesum(den), stage)
    stage = jnp.where(lane == 4, _lanesum(ht), stage)
    vstage[...] = stage
    pltpu.sync_copy(vstage, parts_hbm.at[w])


def _combine_body(parts_ref, ll_ref, lw_ref, ht_ref, cnt_ref):
    p = parts_ref[...]
    sums = p.reshape(_B, 2, 16).sum(axis=1)
    cnt = sums[:, 0]
    ll = sums[:, 1]
    lw_num = sums[:, 2]
    den = sums[:, 3]
    ht = sums[:, 4]
    ll_ref[...] = ll
    lw_ref[...] = cnt * lw_num / den
    ht_ref[...] = ht
    cnt_ref[...] = cnt.astype(jnp.int32)


@jax.jit
def _run(up, uo, h, lam, ts):
    f32 = jnp.float32
    mesh = plsc.VectorSubcoreMesh(core_axis_name="c", subcore_axis_name="s")
    parts = pl.kernel(
        _sc_body,
        out_type=jax.ShapeDtypeStruct((32, 16), f32),
        mesh=mesh,
        scratch_types=[
            pltpu.VMEM((_SLAB,), f32),
            pltpu.VMEM((_SLAB,), f32),
            pltpu.VMEM((16,), f32),
            pltpu.VMEM((16,), f32),
            pltpu.VMEM((16,), f32),
            pltpu.VMEM((16,), f32),
            pltpu.SemaphoreType.DMA,
            pltpu.SemaphoreType.DMA,
        ],
    )(up, uo, h, lam, ts)
    return pl.pallas_call(
        _combine_body,
        out_shape=(
            jax.ShapeDtypeStruct((_B,), f32),
            jax.ShapeDtypeStruct((_B,), f32),
            jax.ShapeDtypeStruct((_B,), f32),
            jax.ShapeDtypeStruct((_B,), jnp.int32),
        ),
    )(parts)


def kernel(u_pred, u_obs, h, lam, thresh_s2_, row_lengths):
    del row_lengths
    up = u_pred.reshape(32, _SLAB)
    uo = u_obs.reshape(32, _SLAB)
    ll, lw, ht, cnt = _run(up, uo, h, lam, thresh_s2_)
    return (ll, lw, ht, cnt)


# speedup vs baseline: 2.4862x; 2.4862x over previous
import functools

import jax
import jax.numpy as jnp
import numpy as np
from jax import lax
from jax.experimental import pallas as pl
from jax.experimental.pallas import tpu as pltpu
from jax.experimental.pallas import tpu_sc as plsc

_SPACE_DIMS = 3
_B = 16
_ROW_LEN = 2048
_N = _B * _ROW_LEN
_HALF = _ROW_LEN // 2
_CHUNKS = _HALF // 16

_THRESH_DEG = 1.0


def _deg2dist(deg):
    return 2.0 * np.sin(np.radians(deg) / 2.0)


_THRESH_S2_MIN = float(_deg2dist(10.0 / 3600.0) ** 2)
_THRESH_S2_MAX = float(_deg2dist(_THRESH_DEG) ** 2)
_LOG_THRESH_S2_RANGE = float(np.log(_THRESH_S2_MAX / _THRESH_S2_MIN))
_THRESH_HIT_PROB_POST = 0.95

_LN2 = np.float32(0.6931471805599453)
_F = jnp.float32


def _approx_sqrt(v):
    vc = jnp.maximum(v, _F(1e-20))
    i = lax.bitcast_convert_type(vc, jnp.int32)
    i = jnp.int32(0x5F3759DF) - lax.shift_right_arithmetic(i, 1)
    y = lax.bitcast_convert_type(i, jnp.float32)
    for _ in range(3):
        y = y * (_F(1.5) - _F(0.5) * vc * y * y)
    return vc * y


def _approx_log(x):
    i = lax.bitcast_convert_type(x, jnp.int32)
    e = lax.shift_right_arithmetic(i, 23) - jnp.int32(127)
    mi = jnp.bitwise_or(jnp.bitwise_and(i, jnp.int32(0x007FFFFF)),
                        jnp.int32(0x3F800000))
    m = lax.bitcast_convert_type(mi, jnp.float32)
    big = m > _F(1.4142135)
    m = jnp.where(big, m * _F(0.5), m)
    e = jnp.where(big, e + 1, e)
    s = (m - _F(1.0)) / (m + _F(1.0))
    s2 = s * s
    poly = _F(1.0) + s2 * (_F(1.0 / 3.0) + s2 * (_F(0.2) + s2 * _F(1.0 / 7.0)))
    return e.astype(jnp.float32) * _LN2 + _F(2.0) * s * poly


def _splat(vec16, idx):
    return vec16[jnp.full((16,), idx, dtype=jnp.int32)]


def _lanesum(x):
    lane = jax.lax.iota(jnp.int32, 16)
    for sh in (8, 4, 2, 1):
        x = x + x[jnp.bitwise_and(lane + sh, 15)]
    return x


def _sc_body(x_hbm, h_hbm, lam_hbm, ts_hbm, parts_hbm,
             vin, vh, vlam, vts, vstage, sem):
    c = lax.axis_index("c")
    s = lax.axis_index("s")
    w = c * 16 + s
    seg = c * 8 + lax.div(s, 2)

    slab = pltpu.async_copy(x_hbm.at[w], vin, sem)
    pltpu.sync_copy(h_hbm, vh)
    pltpu.sync_copy(lam_hbm, vlam)
    pltpu.sync_copy(ts_hbm, vts)

    h16 = vh[...]
    lam16 = vlam[...]
    ts16 = vts[...]
    t16 = _F(_THRESH_S2_MIN) * jnp.exp(ts16 * _F(_LOG_THRESH_S2_RANGE))
    a16 = h16 * lam16 / (_F(1.0) - jnp.exp(-lam16))
    thr = _splat(t16, seg)
    inv_thr = _F(1.0) / thr
    lam_b = _splat(lam16, seg)
    a_b = _splat(a16, seg)
    c_b = _F(1.0) - _splat(h16, seg)
    slab.wait()

    zeros = jnp.zeros((16,), jnp.float32)

    @plsc.parallel_loop(0, _CHUNKS, unroll=4,
                        carry=(zeros, zeros, zeros, zeros, zeros))
    def acc(i, carry):
        cnt, ll, lw, den, ht = carry
        off = i * 16
        px = vin[0, pl.ds(off, 16)]
        py = vin[1, pl.ds(off, 16)]
        pz = vin[2, pl.ds(off, 16)]
        ox = vin[3, pl.ds(off, 16)]
        oy = vin[4, pl.ds(off, 16)]
        oz = vin[5, pl.ds(off, 16)]
        dx = px - ox
        dy = py - oy
        dz = pz - oz
        s2 = dx * dx + dy * dy + dz * dz
        close = s2 < thr
        v = s2 * inv_thr
        obs_w = jnp.where(close, jnp.exp(_F(-2.0) * _approx_sqrt(v)), _F(0.0))
        p_hit = a_b * jnp.exp(-lam_b * v)
        p = p_hit + c_b
        log_p = jnp.where(close, _approx_log(p), _F(0.0))
        php = p_hit / p
        phf = jnp.where(jnp.logical_and(close, php > _F(_THRESH_HIT_PROB_POST)),
                        php, _F(0.0))
        return (cnt + jnp.where(close, _F(1.0), _F(0.0)),
                ll + log_p,
                lw + log_p * obs_w,
                den + obs_w,
                ht + phf)

    cnt, ll, lw, den, ht = acc

    lane = jax.lax.iota(jnp.int32, 16)
    stage = jnp.zeros((16,), jnp.float32)
    stage = jnp.where(lane == 0, _lanesum(cnt), stage)
    stage = jnp.where(lane == 1, _lanesum(ll), stage)
    stage = jnp.where(lane == 2, _lanesum(lw), stage)
    stage = jnp.where(lane == 3, _lanesum(den), stage)
    stage = jnp.where(lane == 4, _lanesum(ht), stage)
    vstage[...] = stage
    pltpu.sync_copy(vstage, parts_hbm.at[w])


def _combine_body(parts_ref, ll_ref, lw_ref, ht_ref, cnt_ref):
    p = parts_ref[...]
    sums = p.reshape(_B, 2, 16).sum(axis=1)
    cnt = sums[:, 0]
    ll = sums[:, 1]
    lw_num = sums[:, 2]
    den = sums[:, 3]
    ht = sums[:, 4]
    ll_ref[...] = ll
    lw_ref[...] = cnt * lw_num / den
    ht_ref[...] = ht
    cnt_ref[...] = cnt.astype(jnp.int32)


@jax.jit
def _run(x, h, lam, ts):
    f32 = jnp.float32
    mesh = plsc.VectorSubcoreMesh(core_axis_name="c", subcore_axis_name="s")
    parts = pl.kernel(
        _sc_body,
        out_type=jax.ShapeDtypeStruct((32, 16), f32),
        mesh=mesh,
        scratch_types=[
            pltpu.VMEM((6, _HALF), f32),
            pltpu.VMEM((16,), f32),
            pltpu.VMEM((16,), f32),
            pltpu.VMEM((16,), f32),
            pltpu.VMEM((16,), f32),
            pltpu.SemaphoreType.DMA,
        ],
    )(x, h, lam, ts)
    return pl.pallas_call(
        _combine_body,
        out_shape=(
            jax.ShapeDtypeStruct((_B,), f32),
            jax.ShapeDtypeStruct((_B,), f32),
            jax.ShapeDtypeStruct((_B,), f32),
            jax.ShapeDtypeStruct((_B,), jnp.int32),
        ),
    )(parts)


def kernel(u_pred, u_obs, h, lam, thresh_s2_, row_lengths):
    del row_lengths
    x = jnp.concatenate([u_pred.T, u_obs.T], axis=0)
    x = x.reshape(6, 32, _HALF).transpose(1, 0, 2)
    ll, lw, ht, cnt = _run(x, h, lam, thresh_s2_)
    return (ll, lw, ht, cnt)


# speedup vs baseline: 2.5146x; 1.0114x over previous
import functools

import jax
import jax.numpy as jnp
import numpy as np
from jax import lax
from jax.experimental import pallas as pl
from jax.experimental.pallas import tpu as pltpu
from jax.experimental.pallas import tpu_sc as plsc

_SPACE_DIMS = 3
_B = 16
_ROW_LEN = 2048
_N = _B * _ROW_LEN
_HALF = _ROW_LEN // 2
_CHUNKS = _HALF // 16

_THRESH_DEG = 1.0


def _deg2dist(deg):
    return 2.0 * np.sin(np.radians(deg) / 2.0)


_THRESH_S2_MIN = float(_deg2dist(10.0 / 3600.0) ** 2)
_THRESH_S2_MAX = float(_deg2dist(_THRESH_DEG) ** 2)
_LOG_THRESH_S2_RANGE = float(np.log(_THRESH_S2_MAX / _THRESH_S2_MIN))
_THRESH_HIT_PROB_POST = 0.95

_LN2 = np.float32(0.6931471805599453)
_F = jnp.float32


def _approx_sqrt(v):
    vc = jnp.maximum(v, _F(1e-20))
    i = lax.bitcast_convert_type(vc, jnp.int32)
    i = jnp.int32(0x5F3759DF) - lax.shift_right_arithmetic(i, 1)
    y = lax.bitcast_convert_type(i, jnp.float32)
    for _ in range(2):
        y = y * (_F(1.5) - _F(0.5) * vc * y * y)
    return vc * y


def _approx_log(x):
    i = lax.bitcast_convert_type(x, jnp.int32)
    e = lax.shift_right_arithmetic(i, 23) - jnp.int32(127)
    mi = jnp.bitwise_or(jnp.bitwise_and(i, jnp.int32(0x007FFFFF)),
                        jnp.int32(0x3F800000))
    m = lax.bitcast_convert_type(mi, jnp.float32)
    big = m > _F(1.4142135)
    m = jnp.where(big, m * _F(0.5), m)
    e = jnp.where(big, e + 1, e)
    s = (m - _F(1.0)) / (m + _F(1.0))
    s2 = s * s
    poly = _F(1.0) + s2 * (_F(1.0 / 3.0) + s2 * (_F(0.2) + s2 * _F(1.0 / 7.0)))
    return e.astype(jnp.float32) * _LN2 + _F(2.0) * s * poly


def _splat(vec16, idx):
    return vec16[jnp.full((16,), idx, dtype=jnp.int32)]


def _lanesum(x):
    lane = jax.lax.iota(jnp.int32, 16)
    for sh in (8, 4, 2, 1):
        x = x + x[jnp.bitwise_and(lane + sh, 15)]
    return x


def _sc_body(x_hbm, h_hbm, lam_hbm, ts_hbm, parts_hbm,
             vin, vh, vlam, vts, vstage, sem):
    c = lax.axis_index("c")
    s = lax.axis_index("s")
    w = c * 16 + s
    seg = c * 8 + lax.div(s, 2)

    slab = pltpu.async_copy(x_hbm.at[w], vin, sem)
    pltpu.sync_copy(h_hbm, vh)
    pltpu.sync_copy(lam_hbm, vlam)
    pltpu.sync_copy(ts_hbm, vts)

    h16 = vh[...]
    lam16 = vlam[...]
    ts16 = vts[...]
    t16 = _F(_THRESH_S2_MIN) * jnp.exp(ts16 * _F(_LOG_THRESH_S2_RANGE))
    a16 = h16 * lam16 / (_F(1.0) - jnp.exp(-lam16))
    thr = _splat(t16, seg)
    inv_thr = _F(1.0) / thr
    lam_b = _splat(lam16, seg)
    a_b = _splat(a16, seg)
    c_b = _F(1.0) - _splat(h16, seg)
    slab.wait()

    zeros = jnp.zeros((16,), jnp.float32)

    @plsc.parallel_loop(0, _CHUNKS, unroll=8,
                        carry=(zeros, zeros, zeros, zeros, zeros))
    def acc(i, carry):
        cnt, ll, lw, den, ht = carry
        off = i * 16
        px = vin[0, pl.ds(off, 16)]
        py = vin[1, pl.ds(off, 16)]
        pz = vin[2, pl.ds(off, 16)]
        ox = vin[3, pl.ds(off, 16)]
        oy = vin[4, pl.ds(off, 16)]
        oz = vin[5, pl.ds(off, 16)]
        dx = px - ox
        dy = py - oy
        dz = pz - oz
        s2 = dx * dx + dy * dy + dz * dz
        close = s2 < thr
        v = s2 * inv_thr
        obs_w = jnp.where(close, jnp.exp(_F(-2.0) * _approx_sqrt(v)), _F(0.0))
        p_hit = a_b * jnp.exp(-lam_b * v)
        p = p_hit + c_b
        log_p = jnp.where(close, _approx_log(p), _F(0.0))
        php = p_hit / p
        phf = jnp.where(jnp.logical_and(close, php > _F(_THRESH_HIT_PROB_POST)),
                        php, _F(0.0))
        return (cnt + jnp.where(close, _F(1.0), _F(0.0)),
                ll + log_p,
                lw + log_p * obs_w,
                den + obs_w,
                ht + phf)

    cnt, ll, lw, den, ht = acc

    lane = jax.lax.iota(jnp.int32, 16)
    stage = jnp.zeros((16,), jnp.float32)
    stage = jnp.where(lane == 0, _lanesum(cnt), stage)
    stage = jnp.where(lane == 1, _lanesum(ll), stage)
    stage = jnp.where(lane == 2, _lanesum(lw), stage)
    stage = jnp.where(lane == 3, _lanesum(den), stage)
    stage = jnp.where(lane == 4, _lanesum(ht), stage)
    vstage[...] = stage
    pltpu.sync_copy(vstage, parts_hbm.at[w])


def _combine_body(parts_ref, ll_ref, lw_ref, ht_ref, cnt_ref):
    p = parts_ref[...]
    sums = p.reshape(_B, 2, 16).sum(axis=1)
    cnt = sums[:, 0]
    ll = sums[:, 1]
    lw_num = sums[:, 2]
    den = sums[:, 3]
    ht = sums[:, 4]
    ll_ref[...] = ll
    lw_ref[...] = cnt * lw_num / den
    ht_ref[...] = ht
    cnt_ref[...] = cnt.astype(jnp.int32)


@jax.jit
def _run(x, h, lam, ts):
    f32 = jnp.float32
    mesh = plsc.VectorSubcoreMesh(core_axis_name="c", subcore_axis_name="s")
    parts = pl.kernel(
        _sc_body,
        out_type=jax.ShapeDtypeStruct((32, 16), f32),
        mesh=mesh,
        scratch_types=[
            pltpu.VMEM((6, _HALF), f32),
            pltpu.VMEM((16,), f32),
            pltpu.VMEM((16,), f32),
            pltpu.VMEM((16,), f32),
            pltpu.VMEM((16,), f32),
            pltpu.SemaphoreType.DMA,
        ],
    )(x, h, lam, ts)
    return pl.pallas_call(
        _combine_body,
        out_shape=(
            jax.ShapeDtypeStruct((_B,), f32),
            jax.ShapeDtypeStruct((_B,), f32),
            jax.ShapeDtypeStruct((_B,), f32),
            jax.ShapeDtypeStruct((_B,), jnp.int32),
        ),
    )(parts)


def kernel(u_pred, u_obs, h, lam, thresh_s2_, row_lengths):
    del row_lengths
    x = jnp.concatenate([u_pred.T, u_obs.T], axis=0)
    x = x.reshape(6, 32, _HALF).transpose(1, 0, 2)
    ll, lw, ht, cnt = _run(x, h, lam, thresh_s2_)
    return (ll, lw, ht, cnt)
